# baseline (speedup 1.0000x reference)
import functools

import jax
import jax.numpy as jnp
from jax import lax
from jax.experimental import pallas as pl
from jax.experimental.pallas import tpu as pltpu
from jax.experimental.pallas import tpu_sc as plsc

_B = 2
_N = 4096
_F48 = 48
_M = 1000
_MCP = 1008
_NSUB = 16
_QPW = _N // _NSUB
_INV_DENOM = 0.08
_NEG_ID = -12.5


def _sc_body(cx, cy, cz, mcx, mcy, mcz, gp, cvx0, cvx1,
             wf, bb, out,
             vx, vy, vz, vxr, vyr, vzr, vsq, vmcx, vmcy, vmcz, vgp,
             vcvx0, vcvx1, vw0, vw1, vwf, vb, vout):
    c = lax.axis_index("c")
    s = lax.axis_index("s")
    base = c * _N
    pltpu.sync_copy(cx.at[pl.ds(base, _N)], vx)
    pltpu.sync_copy(cy.at[pl.ds(base, _N)], vy)
    pltpu.sync_copy(cz.at[pl.ds(base, _N)], vz)
    pltpu.sync_copy(mcx, vmcx)
    pltpu.sync_copy(mcy, vmcy)
    pltpu.sync_copy(mcz, vmcz)
    pltpu.sync_copy(gp, vgp)
    pltpu.sync_copy(cvx0, vcvx0)
    pltpu.sync_copy(cvx1, vcvx1)
    pltpu.sync_copy(wf, vwf)
    pltpu.sync_copy(bb, vb)

    iota = lax.broadcasted_iota(jnp.int32, (16,), 0)

    def _rcp(v):
        bits = lax.bitcast_convert_type(v, jnp.int32)
        y = lax.bitcast_convert_type(
            jnp.full((16,), 0x7EF311C3, jnp.int32) - bits, jnp.float32)
        for _ in range(3):
            y = y * (2.0 - v * y)
        return y

    def _bf16(v):
        bits = lax.bitcast_convert_type(v, jnp.int32)
        half = jnp.full((16,), 0x7FFF, jnp.int32)
        one = jnp.full((16,), 1, jnp.int32)
        mask = jnp.full((16,), -65536, jnp.int32)
        r = (bits + half + ((bits >> 16) & one)) & mask
        return lax.bitcast_convert_type(r, jnp.float32)

    def _sq_body(i, _):
        x = vx[pl.ds(i * 16, 16)]
        y = vy[pl.ds(i * 16, 16)]
        z = vz[pl.ds(i * 16, 16)]
        vsq[pl.ds(i * 16, 16)] = x * x + y * y + z * z
        vxr[pl.ds(i * 16, 16)] = _bf16(x)
        vyr[pl.ds(i * 16, 16)] = _bf16(y)
        vzr[pl.ds(i * 16, 16)] = _bf16(z)
        return 0

    lax.fori_loop(0, _N // 16, _sq_body, 0)

    for strat, (vcvx, vwn, scale) in enumerate(
            (((vcvx0, vw0, 1.0)), (vcvx1, vw1, 0.5))):
        rows = [vcvx[pl.ds(g * 16, 16)] for g in range(8)]
        mx = rows[0]
        for g in range(1, 8):
            mx = jnp.maximum(mx, rows[g])
        es = [jnp.exp(rows[g] - mx) for g in range(8)]
        ssum = es[0]
        for g in range(1, 8):
            ssum = ssum + es[g]
        for g in range(8):
            gxb = plsc.load_gather(vgp, [jnp.full((16,), strat * 24 + g,
                                                  jnp.int32)])
            gyb = plsc.load_gather(vgp, [jnp.full((16,), strat * 24 + 8 + g,
                                                  jnp.int32)])
            gzb = plsc.load_gather(vgp, [jnp.full((16,), strat * 24 + 16 + g,
                                                  jnp.int32)])

            def _mc_body(m, acc, gxb=gxb, gyb=gyb, gzb=gzb):
                ax = vmcx[pl.ds(m * 16, 16)] * gxb
                ay = vmcy[pl.ds(m * 16, 16)] * gyb
                az = vmcz[pl.ds(m * 16, 16)] * gzb
                e = jnp.exp((ax * ax + ay * ay + az * az) * _NEG_ID)
                valid = (m * 16 + iota) < _M
                return acc + jnp.where(valid, e, 0.0)

            acc = lax.fori_loop(0, _MCP // 16, _mc_body,
                                jnp.zeros((16,), jnp.float32))
            integv = jnp.full((16,), jnp.sum(acc) * (1.0 / _M), jnp.float32)
            vwn[pl.ds(g * 16, 16)] = (es[g] * _rcp(ssum)
                                      * _rcp(integv + 1e-8) * scale)

    inf16 = jnp.full((16,), jnp.inf, jnp.float32)

    def _query_body(ql, _):
        qi = s * _QPW + ql
        idxq = jnp.full((16,), qi, jnp.int32)
        qxu = plsc.load_gather(vx, [idxq])
        qyu = plsc.load_gather(vy, [idxq])
        qzu = plsc.load_gather(vz, [idxq])
        qxr = plsc.load_gather(vxr, [idxq])
        qyr = plsc.load_gather(vyr, [idxq])
        qzr = plsc.load_gather(vzr, [idxq])
        sqq = qxu * qxu + qyu * qyu + qzu * qzu

        def _chunk_body(ci, carry):
            kk, vv, t16 = carry
            off = ci * 16
            bx = vxr[pl.ds(off, 16)]
            by = vyr[pl.ds(off, 16)]
            bz = vzr[pl.ds(off, 16)]
            sqs = vsq[pl.ds(off, 16)]
            dot = bx * qxr + by * qyr + bz * qzr
            d2 = sqs + sqq - 2.0 * dot

            def _merge(args):
                kk, vv, _ = args
                ck, cv = plsc.sort_key_val(d2, iota + off)
                rk = lax.rev(ck, (0,))
                rv = lax.rev(cv, (0,))
                sel = rk < kk
                mk = jnp.where(sel, rk, kk)
                mv = jnp.where(sel, rv, vv)
                k2, v2 = plsc.sort_key_val(mk, mv)
                return k2, v2, jnp.full((16,), jnp.max(k2), jnp.float32)

            return lax.cond(jnp.any(d2 < t16), _merge, lambda a: a,
                            (kk, vv, t16))

        kk, vv, _ = lax.fori_loop(
            0, _N // 16, _chunk_body,
            (inf16, jnp.zeros((16,), jnp.int32), inf16))

        nx = plsc.load_gather(vx, [vv])
        ny = plsc.load_gather(vy, [vv])
        nz = plsc.load_gather(vz, [vv])
        relx = nx - qxu
        rely = ny - qyu
        relz = nz - qzu
        m8 = iota < 8

        qoff = ql * _F48
        for strat, vwn in ((0, vw0), (1, vw1)):
            out_acc = jnp.zeros((16,), jnp.float32)
            for g in range(8):
                gxb = plsc.load_gather(vgp, [jnp.full((16,), strat * 24 + g,
                                                      jnp.int32)])
                gyb = plsc.load_gather(vgp, [jnp.full((16,),
                                                      strat * 24 + 8 + g,
                                                      jnp.int32)])
                gzb = plsc.load_gather(vgp, [jnp.full((16,),
                                                      strat * 24 + 16 + g,
                                                      jnp.int32)])
                ax = relx * gxb
                ay = rely * gyb
                az = relz * gzb
                e = jnp.exp((ax * ax + ay * ay + az * az) * _NEG_ID)
                if strat == 0:
                    e = jnp.where(m8, e, 0.0)
                out_acc = out_acc + jnp.sum(e) * vwn[pl.ds(g * 16, 16)]
            vout[pl.ds(qoff + 16 + strat * 16, 16)] = out_acc

        x = vb[...]
        for f in range(32):
            bf = plsc.load_gather(vout, [jnp.full((16,), qoff + 16 + f,
                                                  jnp.int32)])
            x = x + bf * vwf[pl.ds(f * 16, 16)]
        vout[pl.ds(qoff, 16)] = jnp.maximum(x, 0.0)
        return 0

    lax.fori_loop(0, _QPW, _query_body, 0)

    pltpu.sync_copy(vout, out.at[pl.ds((c * _N + s * _QPW) * _F48,
                                       _QPW * _F48)])


def kernel(coords, feats, mc_points, gib_params_0, cvx_0, gib_params_1,
           cvx_1, W, b):
    del feats
    cx, cy, cz = (coords[:, :, d].reshape(-1) for d in range(3))
    mcp = jnp.pad(mc_points, ((0, _MCP - _M), (0, 0)))
    mcx, mcy, mcz = (mcp[:, d] for d in range(3))
    gp = jnp.concatenate([gib_params_0.T.reshape(-1),
                          gib_params_1.T.reshape(-1)])
    cvx0 = cvx_0.T.reshape(-1)
    cvx1 = cvx_1.T.reshape(-1)
    wf = W.reshape(-1)
    mesh = plsc.VectorSubcoreMesh(core_axis_name="c", subcore_axis_name="s")
    run = pl.kernel(
        _sc_body,
        out_type=jax.ShapeDtypeStruct((_B * _N * _F48,), jnp.float32),
        mesh=mesh,
        scratch_types=[
            pltpu.VMEM((_N,), jnp.float32),
            pltpu.VMEM((_N,), jnp.float32),
            pltpu.VMEM((_N,), jnp.float32),
            pltpu.VMEM((_N,), jnp.float32),
            pltpu.VMEM((_N,), jnp.float32),
            pltpu.VMEM((_N,), jnp.float32),
            pltpu.VMEM((_N,), jnp.float32),
            pltpu.VMEM((_MCP,), jnp.float32),
            pltpu.VMEM((_MCP,), jnp.float32),
            pltpu.VMEM((_MCP,), jnp.float32),
            pltpu.VMEM((48,), jnp.float32),
            pltpu.VMEM((128,), jnp.float32),
            pltpu.VMEM((128,), jnp.float32),
            pltpu.VMEM((128,), jnp.float32),
            pltpu.VMEM((128,), jnp.float32),
            pltpu.VMEM((512,), jnp.float32),
            pltpu.VMEM((16,), jnp.float32),
            pltpu.VMEM((_QPW * _F48,), jnp.float32),
        ],
        compiler_params=pltpu.CompilerParams(needs_layout_passes=False),
    )
    flat = run(cx, cy, cz, mcx, mcy, mcz, gp, cvx0, cvx1, wf, b)
    return flat.reshape(_B, _N, _F48)
